# baseline (speedup 1.0000x reference)
import functools

import jax
import jax.numpy as jnp
from jax import lax
from jax.experimental import pallas as pl
from jax.experimental.pallas import tpu as pltpu
from jax.experimental.pallas import tpu_sc as plsc

_N = 10000
_E = 320000
_D = 128

_NC = 2
_NS = 16
_NW = _NC * _NS
_EPT = _E // _NW
_CH = 80
_NCH = _EPT // _CH

_RPT = _N // _NS
_R1 = 624

_mesh = plsc.VectorSubcoreMesh(core_axis_name="c", subcore_axis_name="s")

_GD = lax.GatherDimensionNumbers(
    offset_dims=(), collapsed_slice_dims=(0,), start_index_map=(0,))


def _splat(vec, j):
    idx = jnp.full((16, 1), j, dtype=jnp.int32)
    return lax.gather(vec, idx, _GD, (1,),
                      mode=lax.GatherScatterMode.PROMISE_IN_BOUNDS)


@functools.partial(
    pl.kernel,
    out_type=(jax.ShapeDtypeStruct((_E,), jnp.float32),
              jax.ShapeDtypeStruct((2 * _N,), jnp.float32)),
    mesh=_mesh,
    scratch_types=[
        pltpu.VMEM((_NCH, _CH), jnp.int32),
        pltpu.VMEM((_EPT,), jnp.float32),
        pltpu.VMEM((640,), jnp.float32),
        pltpu.VMEM_SHARED((_N,), jnp.float32),
        pltpu.SemaphoreType.DMA,
    ],
)
def _k0(p_hbm, dst3_hbm, pw_hbm, degp_hbm, idst2, pall, zb, acc, sem):
    cidx = lax.axis_index("c")
    sidx = lax.axis_index("s")
    wid = sidx * _NC + cidx
    r0 = pl.multiple_of(sidx * _R1, 8)
    e0 = pl.multiple_of(wid * _EPT, 8)

    pltpu.sync_copy(dst3_hbm.at[wid], idst2)
    pltpu.sync_copy(p_hbm.at[pl.ds(e0, _EPT)], pall)

    def sig(i, carry):
        v = pall[pl.ds(i * 16, 16)]
        pall[pl.ds(i * 16, 16)] = 1.0 / (1.0 + jnp.exp(-v))
        return carry

    lax.fori_loop(0, _EPT // 16, sig, 0)
    pltpu.sync_copy(pall, pw_hbm.at[pl.ds(e0, _EPT)])

    zv = jnp.zeros((16,), jnp.float32)
    for g in range(640 // 16):
        zb[pl.ds(g * 16, 16)] = zv

    @pl.when(sidx < _NS - 1)
    def _():
        pltpu.sync_copy(zb.at[pl.ds(0, _R1)], acc.at[pl.ds(r0, _R1)])

    @pl.when(sidx == _NS - 1)
    def _():
        pltpu.sync_copy(zb, acc.at[pl.ds(15 * _R1, 640)])

    plsc.subcore_barrier()

    def sc_start(k):
        off = pl.multiple_of(k * _CH, 8)
        pltpu.async_copy(pall.at[pl.ds(off, _CH)], acc.at[idst2.at[k]],
                         sem, add=True)

    def sc_wait(k):
        pltpu.make_async_copy(pall.at[pl.ds(0, _CH)], acc.at[idst2.at[k]],
                              sem).wait()

    def body(k, carry):
        sc_start(k)

        @pl.when(k >= 2)
        def _():
            sc_wait(k - 2)

        return carry

    lax.fori_loop(0, _NCH, body, 0)
    sc_wait(_NCH - 2)
    sc_wait(_NCH - 1)
    plsc.subcore_barrier()

    c0 = pl.multiple_of(cidx * _N, 8)

    @pl.when(sidx < _NS - 1)
    def _():
        pltpu.sync_copy(acc.at[pl.ds(r0, _R1)], zb.at[pl.ds(0, _R1)])
        pltpu.sync_copy(zb.at[pl.ds(0, _R1)], degp_hbm.at[pl.ds(c0 + r0, _R1)])

    @pl.when(sidx == _NS - 1)
    def _():
        pltpu.sync_copy(acc.at[pl.ds(15 * _R1, 640)], zb)
        pltpu.sync_copy(zb, degp_hbm.at[pl.ds(c0 + 15 * _R1, 640)])


@functools.partial(
    pl.kernel,
    out_type=jax.ShapeDtypeStruct((2, _N, _D), jnp.float32),
    mesh=_mesh,
    scratch_types=[
        pltpu.VMEM((_EPT,), jnp.int32),
        pltpu.VMEM((_CH,), jnp.int32),
        pltpu.VMEM((_CH,), jnp.int32),
        pltpu.VMEM((_CH,), jnp.int32),
        pltpu.VMEM((_CH,), jnp.float32),
        pltpu.VMEM((_CH,), jnp.float32),
        pltpu.VMEM((_CH,), jnp.float32),
        pltpu.VMEM((_CH, _D), jnp.float32),
        pltpu.VMEM((_CH, _D), jnp.float32),
        pltpu.VMEM((_CH, _D), jnp.float32),
        pltpu.VMEM_SHARED((_N, _D), jnp.float32),
        pltpu.SemaphoreType.DMA,
        pltpu.SemaphoreType.DMA,
        pltpu.SemaphoreType.DMA,
        pltpu.SemaphoreType.DMA,
        pltpu.SemaphoreType.DMA,
        pltpu.SemaphoreType.DMA,
    ],
)
def _kagg(hp_hbm, src_hbm, dst_hbm, pw_hbm, out_hbm,
          isrc, id0, id1, id2, pw0, pw1, pw2, rw0, rw1, rw2, acc,
          sg0, sg1, sg2, ss0, ss1, ss2):
    cidx = lax.axis_index("c")
    sidx = lax.axis_index("s")
    wid = sidx * _NC + cidx
    r0 = pl.multiple_of(sidx * _R1, 8)
    e0 = pl.multiple_of(wid * _EPT, 8)

    idst = (id0, id1, id2)
    pwb = (pw0, pw1, pw2)
    rows = (rw0, rw1, rw2)
    sg = (sg0, sg1, sg2)
    ss = (ss0, ss1, ss2)

    pltpu.sync_copy(src_hbm.at[pl.ds(e0, _EPT)], isrc)

    zv = jnp.zeros((16,), jnp.float32)
    for e in range(_CH):
        for f in range(_D // 16):
            rw0[e, pl.ds(f * 16, 16)] = zv

    @pl.when(sidx < _NS - 1)
    def _():
        for q in range(7):
            pltpu.sync_copy(rw0, acc.at[pl.ds(r0 + q * _CH, _CH), :])
        pltpu.sync_copy(rw0.at[pl.ds(0, 64), :],
                        acc.at[pl.ds(r0 + 560, 64), :])

    @pl.when(sidx == _NS - 1)
    def _():
        for q in range(8):
            pltpu.sync_copy(rw0, acc.at[pl.ds(15 * _R1 + q * _CH, _CH), :])

    plsc.subcore_barrier()

    def start_gather(k, b):
        off = pl.multiple_of(k * _CH, 8)
        base = pl.multiple_of(wid * _EPT + k * _CH, 8)
        pltpu.async_copy(pw_hbm.at[pl.ds(base, _CH)], pwb[b], sg[b])
        pltpu.async_copy(dst_hbm.at[pl.ds(base, _CH)], idst[b], sg[b])
        pltpu.async_copy(hp_hbm.at[isrc.at[pl.ds(off, _CH)]], rows[b], sg[b])

    def wait_gather(b):
        pltpu.make_async_copy(pw_hbm.at[pl.ds(0, _CH)], pwb[b], sg[b]).wait()
        pltpu.make_async_copy(dst_hbm.at[pl.ds(0, _CH)], idst[b], sg[b]).wait()
        pltpu.make_async_copy(hp_hbm.at[isrc.at[pl.ds(0, _CH)]], rows[b],
                              sg[b]).wait()

    def scale(b):
        for g in range(_CH // 16):
            pv = pwb[b][pl.ds(g * 16, 16)]
            for j in range(16):
                e = g * 16 + j
                s = _splat(pv, j)
                for f in range(_D // 16):
                    rows[b][e, pl.ds(f * 16, 16)] = (
                        rows[b][e, pl.ds(f * 16, 16)] * s)

    def start_scatter(b):
        pltpu.async_copy(rows[b], acc.at[idst[b]], ss[b], add=True)

    def wait_scatter(b):
        pltpu.make_async_copy(rows[b], acc.at[idst[b]], ss[b]).wait()

    start_gather(0, 0)

    def body(m, carry):
        for d in range(3):
            k = m * 3 + d
            nxt = (d + 1) % 3

            @pl.when(k >= 2)
            def _():
                wait_scatter(nxt)

            start_gather(k + 1, nxt)
            wait_gather(d)
            scale(d)
            start_scatter(d)
        return carry

    lax.fori_loop(0, (_NCH - 2) // 3, body, 0)
    wait_scatter(1)
    start_gather(_NCH - 1, 1)
    wait_gather(0)
    scale(0)
    start_scatter(0)
    wait_scatter(2)
    wait_gather(1)
    scale(1)
    start_scatter(1)
    wait_scatter(0)
    wait_scatter(1)
    plsc.subcore_barrier()

    @pl.when(sidx < _NS - 1)
    def _():
        for q in range(7):
            pltpu.sync_copy(acc.at[pl.ds(r0 + q * _CH, _CH), :], rw0)
            pltpu.sync_copy(rw0, out_hbm.at[cidx, pl.ds(r0 + q * _CH, _CH), :])
        pltpu.sync_copy(acc.at[pl.ds(r0 + 560, 64), :],
                        rw0.at[pl.ds(0, 64), :])
        pltpu.sync_copy(rw0.at[pl.ds(0, 64), :],
                        out_hbm.at[cidx, pl.ds(r0 + 560, 64), :])

    @pl.when(sidx == _NS - 1)
    def _():
        for q in range(8):
            pltpu.sync_copy(acc.at[pl.ds(15 * _R1 + q * _CH, _CH), :], rw0)
            pltpu.sync_copy(rw0, out_hbm.at[cidx,
                                            pl.ds(15 * _R1 + q * _CH, _CH), :])


_BN = 2000


def _ka_body(x_ref, w_ref, deg_ref, o_ref):
    dis = lax.rsqrt(deg_ref[...])
    h = lax.dot_general(x_ref[...], w_ref[...], (((1,), (1,)), ((), ())),
                        preferred_element_type=jnp.float32)
    o_ref[...] = h * dis


def _ka(x, w1, deg2):
    return pl.pallas_call(
        _ka_body,
        grid=(_N // _BN,),
        in_specs=[pl.BlockSpec((_BN, _D), lambda i: (i, 0)),
                  pl.BlockSpec((_D, _D), lambda i: (0, 0)),
                  pl.BlockSpec((_BN, 1), lambda i: (i, 0))],
        out_specs=pl.BlockSpec((_BN, _D), lambda i: (i, 0)),
        out_shape=jax.ShapeDtypeStruct((_N, _D), jnp.float32),
    )(x, w1, deg2)


def _kb_body(agg_ref, hp_ref, deg_ref, b1_ref, w2_ref, o_ref):
    dis = lax.rsqrt(deg_ref[...])
    a = agg_ref[0] + agg_ref[1] + hp_ref[...]
    y = jnp.maximum(a * dis + b1_ref[...], 0.0)
    h2 = lax.dot_general(y, w2_ref[...], (((1,), (1,)), ((), ())),
                         preferred_element_type=jnp.float32)
    o_ref[...] = h2 * dis


def _kb(aggp, hp, deg2, b1, w2):
    return pl.pallas_call(
        _kb_body,
        grid=(_N // _BN,),
        in_specs=[pl.BlockSpec((2, _BN, _D), lambda i: (0, i, 0)),
                  pl.BlockSpec((_BN, _D), lambda i: (i, 0)),
                  pl.BlockSpec((_BN, 1), lambda i: (i, 0)),
                  pl.BlockSpec((1, _D), lambda i: (0, 0)),
                  pl.BlockSpec((_D, _D), lambda i: (0, 0))],
        out_specs=pl.BlockSpec((_BN, _D), lambda i: (i, 0)),
        out_shape=jax.ShapeDtypeStruct((_N, _D), jnp.float32),
    )(aggp, hp, deg2, b1, w2)


def _kc_body(agg_ref, hp_ref, deg_ref, b2_ref, o_ref):
    dis = lax.rsqrt(deg_ref[...])
    a = agg_ref[0] + agg_ref[1] + hp_ref[...]
    o_ref[...] = a * dis + b2_ref[...]


def _kc(aggp, hp, deg2, b2):
    return pl.pallas_call(
        _kc_body,
        grid=(_N // _BN,),
        in_specs=[pl.BlockSpec((2, _BN, _D), lambda i: (0, i, 0)),
                  pl.BlockSpec((_BN, _D), lambda i: (i, 0)),
                  pl.BlockSpec((_BN, 1), lambda i: (i, 0)),
                  pl.BlockSpec((1, _D), lambda i: (0, 0))],
        out_specs=pl.BlockSpec((_BN, _D), lambda i: (i, 0)),
        out_shape=jax.ShapeDtypeStruct((_N, _D), jnp.float32),
    )(aggp, hp, deg2, b2)


def kernel(x, edge_index, P_vec, W1, b1, W2, b2):
    ei = edge_index.astype(jnp.int32)
    src = ei[0]
    dst = ei[1]
    dst3 = dst.reshape(_NW, _NCH, _CH)
    pw, degp = _k0(P_vec, dst3)
    deg2 = (degp[:_N] + degp[_N:] + 1.0)[:, None]
    h1p = _ka(x, W1, deg2)
    agg1 = _kagg(h1p, src, dst, pw)
    h2p = _kb(agg1, h1p, deg2, b1.reshape(1, _D), W2)
    agg2 = _kagg(h2p, src, dst, pw)
    return _kc(agg2, h2p, deg2, b2.reshape(1, _D))


# speedup vs baseline: 1.0352x; 1.0352x over previous
import functools

import jax
import jax.numpy as jnp
from jax import lax
from jax.experimental import pallas as pl
from jax.experimental.pallas import tpu as pltpu
from jax.experimental.pallas import tpu_sc as plsc

_N = 10000
_E = 320000
_D = 128

_NC = 2
_NS = 16
_NW = _NC * _NS
_EPT = _E // _NW
_CH = 80
_NCH = _EPT // _CH

_RPT = _N // _NS
_R1 = 624

_mesh = plsc.VectorSubcoreMesh(core_axis_name="c", subcore_axis_name="s")

_GD = lax.GatherDimensionNumbers(
    offset_dims=(), collapsed_slice_dims=(0,), start_index_map=(0,))


def _splat(vec, j):
    idx = jnp.full((16, 1), j, dtype=jnp.int32)
    return lax.gather(vec, idx, _GD, (1,),
                      mode=lax.GatherScatterMode.PROMISE_IN_BOUNDS)


@functools.partial(
    pl.kernel,
    out_type=(jax.ShapeDtypeStruct((_E,), jnp.float32),
              jax.ShapeDtypeStruct((2 * _N,), jnp.float32)),
    mesh=_mesh,
    scratch_types=[
        pltpu.VMEM((_NCH, _CH), jnp.int32),
        pltpu.VMEM((_EPT,), jnp.float32),
        pltpu.VMEM((640,), jnp.float32),
        pltpu.VMEM_SHARED((_N,), jnp.float32),
        pltpu.SemaphoreType.DMA,
    ],
)
def _k0(p_hbm, dst3_hbm, pw_hbm, degp_hbm, idst2, pall, zb, acc, sem):
    cidx = lax.axis_index("c")
    sidx = lax.axis_index("s")
    wid = sidx * _NC + cidx
    r0 = pl.multiple_of(sidx * _R1, 8)
    e0 = pl.multiple_of(wid * _EPT, 8)

    pltpu.sync_copy(dst3_hbm.at[wid], idst2)
    pltpu.sync_copy(p_hbm.at[pl.ds(e0, _EPT)], pall)

    def sig(i, carry):
        v = pall[pl.ds(i * 16, 16)]
        pall[pl.ds(i * 16, 16)] = 1.0 / (1.0 + jnp.exp(-v))
        return carry

    lax.fori_loop(0, _EPT // 16, sig, 0)
    pltpu.sync_copy(pall, pw_hbm.at[pl.ds(e0, _EPT)])

    zv = jnp.zeros((16,), jnp.float32)
    for g in range(640 // 16):
        zb[pl.ds(g * 16, 16)] = zv

    @pl.when(sidx < _NS - 1)
    def _():
        pltpu.sync_copy(zb.at[pl.ds(0, _R1)], acc.at[pl.ds(r0, _R1)])

    @pl.when(sidx == _NS - 1)
    def _():
        pltpu.sync_copy(zb, acc.at[pl.ds(15 * _R1, 640)])

    plsc.subcore_barrier()

    def sc_start(k):
        off = pl.multiple_of(k * _CH, 8)
        pltpu.async_copy(pall.at[pl.ds(off, _CH)], acc.at[idst2.at[k]],
                         sem, add=True)

    def sc_wait(k):
        pltpu.make_async_copy(pall.at[pl.ds(0, _CH)], acc.at[idst2.at[k]],
                              sem).wait()

    def body(k, carry):
        sc_start(k)

        @pl.when(k >= 2)
        def _():
            sc_wait(k - 2)

        return carry

    lax.fori_loop(0, _NCH, body, 0)
    sc_wait(_NCH - 2)
    sc_wait(_NCH - 1)
    plsc.subcore_barrier()

    c0 = pl.multiple_of(cidx * _N, 8)

    @pl.when(sidx < _NS - 1)
    def _():
        pltpu.sync_copy(acc.at[pl.ds(r0, _R1)], zb.at[pl.ds(0, _R1)])
        pltpu.sync_copy(zb.at[pl.ds(0, _R1)], degp_hbm.at[pl.ds(c0 + r0, _R1)])

    @pl.when(sidx == _NS - 1)
    def _():
        pltpu.sync_copy(acc.at[pl.ds(15 * _R1, 640)], zb)
        pltpu.sync_copy(zb, degp_hbm.at[pl.ds(c0 + 15 * _R1, 640)])


@functools.partial(
    pl.kernel,
    out_type=jax.ShapeDtypeStruct((2, _N, _D), jnp.float32),
    mesh=_mesh,
    scratch_types=[
        pltpu.VMEM((_EPT,), jnp.int32),
        pltpu.VMEM((_NCH, _CH), jnp.int32),
        pltpu.VMEM((_CH,), jnp.float32),
        pltpu.VMEM((_CH,), jnp.float32),
        pltpu.VMEM((_CH, _D), jnp.float32),
        pltpu.VMEM((_CH, _D), jnp.float32),
        pltpu.VMEM_SHARED((_N, _D), jnp.float32),
        pltpu.SemaphoreType.DMA,
        pltpu.SemaphoreType.DMA,
        pltpu.SemaphoreType.DMA,
        pltpu.SemaphoreType.DMA,
    ],
)
def _kagg(hp_hbm, src_hbm, dst3_hbm, pw_hbm, out_hbm,
          isrc, idst2, pwb_a, pwb_b, rows_a, rows_b, acc, sga, sgb, ssa, ssb):
    cidx = lax.axis_index("c")
    sidx = lax.axis_index("s")
    wid = sidx * _NC + cidx
    r0 = pl.multiple_of(sidx * _R1, 8)
    e0 = pl.multiple_of(wid * _EPT, 8)

    pltpu.sync_copy(src_hbm.at[pl.ds(e0, _EPT)], isrc)
    pltpu.sync_copy(dst3_hbm.at[wid], idst2)

    zv = jnp.zeros((16,), jnp.float32)
    for e in range(_CH):
        for f in range(_D // 16):
            rows_a[e, pl.ds(f * 16, 16)] = zv

    @pl.when(sidx < _NS - 1)
    def _():
        for q in range(7):
            pltpu.sync_copy(rows_a, acc.at[pl.ds(r0 + q * _CH, _CH), :])
        pltpu.sync_copy(rows_a.at[pl.ds(0, 64), :],
                        acc.at[pl.ds(r0 + 560, 64), :])

    @pl.when(sidx == _NS - 1)
    def _():
        for q in range(8):
            pltpu.sync_copy(rows_a, acc.at[pl.ds(15 * _R1 + q * _CH, _CH), :])

    plsc.subcore_barrier()

    def start_gather(k, rows, pwb, sem):
        off = pl.multiple_of(k * _CH, 8)
        base = pl.multiple_of(wid * _EPT + k * _CH, 8)
        pltpu.async_copy(pw_hbm.at[pl.ds(base, _CH)], pwb, sem)
        pltpu.async_copy(hp_hbm.at[isrc.at[pl.ds(off, _CH)]], rows, sem)

    def wait_gather(rows, pwb, sem):
        pltpu.make_async_copy(pw_hbm.at[pl.ds(0, _CH)], pwb, sem).wait()
        pltpu.make_async_copy(hp_hbm.at[isrc.at[pl.ds(0, _CH)]], rows,
                              sem).wait()

    def scale(pwb, rows):
        for g in range(_CH // 16):
            pv = pwb[pl.ds(g * 16, 16)]
            for j in range(16):
                e = g * 16 + j
                s = _splat(pv, j)
                for f in range(_D // 16):
                    rows[e, pl.ds(f * 16, 16)] = rows[e, pl.ds(f * 16, 16)] * s

    def start_scatter(k, rows, sem):
        pltpu.async_copy(rows, acc.at[idst2.at[k]], sem, add=True)

    def wait_scatter(rows, sem):
        pltpu.make_async_copy(rows, acc.at[idst2.at[0]], sem).wait()

    start_gather(0, rows_a, pwb_a, sga)

    def body(m, carry):
        k0 = m * 2

        @pl.when(m > 0)
        def _():
            wait_scatter(rows_b, ssb)

        start_gather(k0 + 1, rows_b, pwb_b, sgb)
        wait_gather(rows_a, pwb_a, sga)
        scale(pwb_a, rows_a)
        start_scatter(k0, rows_a, ssa)
        wait_gather(rows_b, pwb_b, sgb)
        scale(pwb_b, rows_b)
        wait_scatter(rows_a, ssa)
        start_gather(k0 + 2, rows_a, pwb_a, sga)
        start_scatter(k0 + 1, rows_b, ssb)
        return carry

    lax.fori_loop(0, (_NCH - 1) // 2, body, 0)
    wait_scatter(rows_b, ssb)
    wait_gather(rows_a, pwb_a, sga)
    scale(pwb_a, rows_a)
    start_scatter(_NCH - 1, rows_a, ssa)
    wait_scatter(rows_a, ssa)
    plsc.subcore_barrier()

    @pl.when(sidx < _NS - 1)
    def _():
        for q in range(7):
            pltpu.sync_copy(acc.at[pl.ds(r0 + q * _CH, _CH), :], rows_a)
            pltpu.sync_copy(rows_a, out_hbm.at[cidx, pl.ds(r0 + q * _CH, _CH), :])
        pltpu.sync_copy(acc.at[pl.ds(r0 + 560, 64), :],
                        rows_a.at[pl.ds(0, 64), :])
        pltpu.sync_copy(rows_a.at[pl.ds(0, 64), :],
                        out_hbm.at[cidx, pl.ds(r0 + 560, 64), :])

    @pl.when(sidx == _NS - 1)
    def _():
        for q in range(8):
            pltpu.sync_copy(acc.at[pl.ds(15 * _R1 + q * _CH, _CH), :], rows_a)
            pltpu.sync_copy(rows_a, out_hbm.at[cidx,
                                               pl.ds(15 * _R1 + q * _CH, _CH), :])


_BN = 2000


def _ka_body(x_ref, w_ref, deg_ref, o_ref):
    dis = lax.rsqrt(deg_ref[...])
    h = lax.dot_general(x_ref[...], w_ref[...], (((1,), (1,)), ((), ())),
                        preferred_element_type=jnp.float32)
    o_ref[...] = h * dis


def _ka(x, w1, deg2):
    return pl.pallas_call(
        _ka_body,
        grid=(_N // _BN,),
        in_specs=[pl.BlockSpec((_BN, _D), lambda i: (i, 0)),
                  pl.BlockSpec((_D, _D), lambda i: (0, 0)),
                  pl.BlockSpec((_BN, 1), lambda i: (i, 0))],
        out_specs=pl.BlockSpec((_BN, _D), lambda i: (i, 0)),
        out_shape=jax.ShapeDtypeStruct((_N, _D), jnp.float32),
    )(x, w1, deg2)


def _kb_body(agg_ref, hp_ref, deg_ref, b1_ref, w2_ref, o_ref):
    dis = lax.rsqrt(deg_ref[...])
    a = agg_ref[0] + agg_ref[1] + hp_ref[...]
    y = jnp.maximum(a * dis + b1_ref[...], 0.0)
    h2 = lax.dot_general(y, w2_ref[...], (((1,), (1,)), ((), ())),
                         preferred_element_type=jnp.float32)
    o_ref[...] = h2 * dis


def _kb(aggp, hp, deg2, b1, w2):
    return pl.pallas_call(
        _kb_body,
        grid=(_N // _BN,),
        in_specs=[pl.BlockSpec((2, _BN, _D), lambda i: (0, i, 0)),
                  pl.BlockSpec((_BN, _D), lambda i: (i, 0)),
                  pl.BlockSpec((_BN, 1), lambda i: (i, 0)),
                  pl.BlockSpec((1, _D), lambda i: (0, 0)),
                  pl.BlockSpec((_D, _D), lambda i: (0, 0))],
        out_specs=pl.BlockSpec((_BN, _D), lambda i: (i, 0)),
        out_shape=jax.ShapeDtypeStruct((_N, _D), jnp.float32),
    )(aggp, hp, deg2, b1, w2)


def _kc_body(agg_ref, hp_ref, deg_ref, b2_ref, o_ref):
    dis = lax.rsqrt(deg_ref[...])
    a = agg_ref[0] + agg_ref[1] + hp_ref[...]
    o_ref[...] = a * dis + b2_ref[...]


def _kc(aggp, hp, deg2, b2):
    return pl.pallas_call(
        _kc_body,
        grid=(_N // _BN,),
        in_specs=[pl.BlockSpec((2, _BN, _D), lambda i: (0, i, 0)),
                  pl.BlockSpec((_BN, _D), lambda i: (i, 0)),
                  pl.BlockSpec((_BN, 1), lambda i: (i, 0)),
                  pl.BlockSpec((1, _D), lambda i: (0, 0))],
        out_specs=pl.BlockSpec((_BN, _D), lambda i: (i, 0)),
        out_shape=jax.ShapeDtypeStruct((_N, _D), jnp.float32),
    )(aggp, hp, deg2, b2)


def kernel(x, edge_index, P_vec, W1, b1, W2, b2):
    ei = edge_index.astype(jnp.int32)
    src = ei[0]
    dst3 = ei[1].reshape(_NW, _NCH, _CH)
    pw, degp = _k0(P_vec, dst3)
    deg2 = (degp[:_N] + degp[_N:] + 1.0)[:, None]
    h1p = _ka(x, W1, deg2)
    agg1 = _kagg(h1p, src, dst3, pw)
    h2p = _kb(agg1, h1p, deg2, b1.reshape(1, _D), W2)
    agg2 = _kagg(h2p, src, dst3, pw)
    return _kc(agg2, h2p, deg2, b2.reshape(1, _D))
